# baseline (speedup 1.0000x reference)
import jax
import jax.numpy as jnp
from jax import lax
from jax.experimental import pallas as pl
from jax.experimental.pallas import tpu as pltpu
from jax.experimental.pallas import tpu_sc as plsc

_F = 128
_NPIDS = 100000
_NCQ = 5000
_SCALE = 30.0
_B = 1024
_LOG2E = 1.4426950408889634
_LN2 = 0.6931471805599453

_NL_TILE = 10000
_KL = _NPIDS // _NL_TILE
_NC_TILE = 5000
_KC = _NCQ // _NC_TILE
_K = _KL + _KC

_NW = 32
_BPW = _B // _NW


def _sc_gather_body(table_hbm, idx_hbm, out_hbm, idx_v, rows_v, sem):
    wid = lax.axis_index("s") * 2 + lax.axis_index("c")
    base = wid * _BPW
    pltpu.sync_copy(idx_hbm.at[pl.ds(base, _BPW)], idx_v)
    pltpu.async_copy(table_hbm.at[idx_v], rows_v, sem).wait()
    pltpu.sync_copy(rows_v, out_hbm.at[pl.ds(base, _BPW)])


def _gather_label_rows(lut, idx):
    mesh = plsc.VectorSubcoreMesh(core_axis_name="c", subcore_axis_name="s")
    return pl.kernel(
        _sc_gather_body,
        mesh=mesh,
        out_type=jax.ShapeDtypeStruct((_B, _F), jnp.float32),
        scratch_types=[
            pltpu.VMEM((_BPW,), jnp.int32),
            pltpu.VMEM((_BPW, _F), jnp.float32),
            pltpu.SemaphoreType.DMA,
        ],
    )(lut, idx)


def _body(x_ref, lab_ref, iou_ref, g_ref, lut_ref, cq_ref, out_ref,
          xh_ref, xh16_ref, m_ref, s_ref, nb_ref):
    k = pl.program_id(0)

    @pl.when(k == 0)
    def _init():
        x = x_ref[...]
        ss = jnp.sum(x * x, axis=1, keepdims=True)
        nrm = jnp.maximum(jnp.sqrt(ss), 1e-12)
        xh_ref[...] = x * (_SCALE / nrm)
        xh16_ref[...] = (xh_ref[...] * _LOG2E).astype(jnp.bfloat16)
        m_ref[...] = jnp.full((_B, 1), -1e30, jnp.float32)
        s_ref[...] = jnp.zeros((_B, 1), jnp.float32)
        nb_ref[...] = jnp.zeros((1, 1), jnp.float32)

    def _process(tbl):
        tbl16 = tbl.astype(jnp.bfloat16)
        ones = jnp.ones((1, _F), jnp.bfloat16)
        q = jax.lax.dot_general(ones, tbl16 * tbl16,
                                (((1,), (1,)), ((), ())),
                                preferred_element_type=jnp.float32)
        nb_ref[...] += jnp.sum(jnp.where(q == 0.0, 1.0, 0.0),
                               axis=(0, 1), keepdims=True)
        bound = (_SCALE * _LOG2E * 1.01) * jnp.sqrt(
            jnp.max(q, axis=(0, 1), keepdims=True))
        p = jax.lax.dot_general(xh16_ref[...], tbl16,
                                (((1,), (1,)), ((), ())),
                                preferred_element_type=jnp.float32)
        m_old = m_ref[...]
        m_new = jnp.maximum(m_old, bound)
        e = jnp.exp2((p - m_new).astype(jnp.bfloat16))
        s_ref[...] = (s_ref[...] * jnp.exp2(m_old - m_new)
                      + jnp.sum(e.astype(jnp.float32), axis=1, keepdims=True))
        m_ref[...] = m_new

    @pl.when(k < _KL)
    def _lut_step():
        _process(lut_ref[...])

    @pl.when(k >= _KL)
    def _cq_step():
        _process(cq_ref[...])

    @pl.when(k == _K - 1)
    def _finish():
        lab = lab_ref[...]
        iou = iou_ref[...]
        g = g_ref[...]
        xh = xh_ref[...]
        d = jnp.sum(xh * g, axis=1, keepdims=True)
        gz = jnp.sum(jnp.abs(g), axis=1, keepdims=True)
        bad_l = gz == 0.0
        a = jnp.where(bad_l, -_SCALE, d)
        b = jnp.where(bad_l, _SCALE, d)
        m_raw = m_ref[...] * _LN2
        nb = nb_ref[...]
        s_raw = ((s_ref[...] - nb * jnp.exp(jnp.minimum(-m_raw, 0.0)))
                 + nb * jnp.exp(jnp.minimum(-_SCALE - m_raw, 0.0)))
        m2 = jnp.maximum(m_raw, b)
        s2 = (s_raw * jnp.exp(m_raw - m2)
              - jnp.exp(a - m2) + jnp.exp(b - m2))
        lse = m2 + jnp.log(s2)
        valid = lab < _NPIDS
        terms = jnp.where(valid, iou * iou * (lse - b), 0.0)
        nvalid = jnp.sum(jnp.where(valid, 1.0, 0.0), axis=(0, 1),
                         keepdims=True)
        out_ref[...] = jnp.sum(terms, axis=(0, 1), keepdims=True) / nvalid


def kernel(inputs, label, iou, lut, cq):
    lab_safe = jnp.clip(label, 0, _NPIDS - 1)
    g = _gather_label_rows(lut, lab_safe)
    lab2 = label.reshape(_B, 1)
    iou2 = iou.reshape(_B, 1)
    out = pl.pallas_call(
        _body,
        grid=(_K,),
        in_specs=[
            pl.BlockSpec((_B, _F), lambda k: (0, 0)),
            pl.BlockSpec((_B, 1), lambda k: (0, 0)),
            pl.BlockSpec((_B, 1), lambda k: (0, 0)),
            pl.BlockSpec((_B, _F), lambda k: (0, 0)),
            pl.BlockSpec((_NL_TILE, _F),
                         lambda k: (jnp.minimum(k, _KL - 1), 0)),
            pl.BlockSpec((_NC_TILE, _F),
                         lambda k: (jnp.maximum(k - _KL, 0), 0)),
        ],
        out_specs=pl.BlockSpec((1, 1), lambda k: (0, 0)),
        out_shape=jax.ShapeDtypeStruct((1, 1), jnp.float32),
        scratch_shapes=[
            pltpu.VMEM((_B, _F), jnp.float32),
            pltpu.VMEM((_B, _F), jnp.bfloat16),
            pltpu.VMEM((_B, 1), jnp.float32),
            pltpu.VMEM((_B, 1), jnp.float32),
            pltpu.VMEM((1, 1), jnp.float32),
        ],
        compiler_params=pltpu.CompilerParams(
            dimension_semantics=("arbitrary",),
        ),
    )(inputs, lab2, iou2, g, lut, cq)
    return out[0, 0]


# speedup vs baseline: 1.0360x; 1.0360x over previous
import jax
import jax.numpy as jnp
from jax import lax
from jax.experimental import pallas as pl
from jax.experimental.pallas import tpu as pltpu
from jax.experimental.pallas import tpu_sc as plsc

_F = 128
_NPIDS = 100000
_NCQ = 5000
_SCALE = 30.0
_B = 1024
_LOG2E = 1.4426950408889634
_LN2 = 0.6931471805599453

_NL_TILE = 10000
_KL = _NPIDS // _NL_TILE
_NC_TILE = 5000
_KC = _NCQ // _NC_TILE
_K = _KL + _KC

_NW = 32
_BPW = _B // _NW


def _sc_gather_body(table_hbm, idx_hbm, out_hbm, idx_v, rows_v, sem):
    wid = lax.axis_index("s") * 2 + lax.axis_index("c")
    base = wid * _BPW
    pltpu.sync_copy(idx_hbm.at[pl.ds(base, _BPW)], idx_v)
    pltpu.async_copy(table_hbm.at[idx_v], rows_v, sem).wait()
    pltpu.sync_copy(rows_v, out_hbm.at[pl.ds(base, _BPW)])


def _gather_label_rows(lut, idx):
    mesh = plsc.VectorSubcoreMesh(core_axis_name="c", subcore_axis_name="s")
    return pl.kernel(
        _sc_gather_body,
        mesh=mesh,
        out_type=jax.ShapeDtypeStruct((_B, _F), jnp.float32),
        scratch_types=[
            pltpu.VMEM((_BPW,), jnp.int32),
            pltpu.VMEM((_BPW, _F), jnp.float32),
            pltpu.SemaphoreType.DMA,
        ],
    )(lut, idx)


def _stream_body(x_ref, lut_ref, cq_ref, m_ref, s_ref, nb_ref, xh16_ref):
    k = pl.program_id(0)

    @pl.when(k == 0)
    def _init():
        x = x_ref[...]
        ss = jnp.sum(x * x, axis=1, keepdims=True)
        nrm = jnp.maximum(jnp.sqrt(ss), 1e-12)
        xh16_ref[...] = (x * (_SCALE * _LOG2E / nrm)).astype(jnp.bfloat16)
        m_ref[...] = jnp.full((_B, 1), -1e30, jnp.float32)
        s_ref[...] = jnp.zeros((_B, 1), jnp.float32)
        nb_ref[...] = jnp.zeros((1, 1), jnp.float32)

    def _process(tbl):
        tbl16 = tbl.astype(jnp.bfloat16)
        ones = jnp.ones((1, _F), jnp.bfloat16)
        q = jax.lax.dot_general(ones, tbl16 * tbl16,
                                (((1,), (1,)), ((), ())),
                                preferred_element_type=jnp.float32)
        nb_ref[...] += jnp.sum(jnp.where(q == 0.0, 1.0, 0.0),
                               axis=(0, 1), keepdims=True)
        bound = (_SCALE * _LOG2E * 1.01) * jnp.sqrt(
            jnp.max(q, axis=(0, 1), keepdims=True))
        p = jax.lax.dot_general(xh16_ref[...], tbl16,
                                (((1,), (1,)), ((), ())),
                                preferred_element_type=jnp.float32)
        m_old = m_ref[...]
        m_new = jnp.maximum(m_old, bound)
        s_ref[...] = (s_ref[...] * jnp.exp2(m_old - m_new)
                      + jnp.sum(jnp.exp2(p - m_new), axis=1, keepdims=True))
        m_ref[...] = m_new

    @pl.when(k < _KL)
    def _lut_step():
        _process(lut_ref[...])

    @pl.when(k >= _KL)
    def _cq_step():
        _process(cq_ref[...])


def _final_body(x_ref, lab_ref, iou_ref, g_ref, m_ref, s_ref, nb_ref,
                out_ref):
    x = x_ref[...]
    ss = jnp.sum(x * x, axis=1, keepdims=True)
    nrm = jnp.maximum(jnp.sqrt(ss), 1e-12)
    xh = x * (_SCALE / nrm)
    lab = lab_ref[...]
    iou = iou_ref[...]
    g = g_ref[...]
    d = jnp.sum(xh * g, axis=1, keepdims=True)
    gz = jnp.sum(jnp.abs(g), axis=1, keepdims=True)
    bad_l = gz == 0.0
    a = jnp.where(bad_l, -_SCALE, d)
    b = jnp.where(bad_l, _SCALE, d)
    m_raw = m_ref[...] * _LN2
    nb = nb_ref[...]
    s_raw = ((s_ref[...] - nb * jnp.exp(jnp.minimum(-m_raw, 0.0)))
             + nb * jnp.exp(jnp.minimum(-_SCALE - m_raw, 0.0)))
    m2 = jnp.maximum(m_raw, b)
    s2 = (s_raw * jnp.exp(m_raw - m2)
          - jnp.exp(a - m2) + jnp.exp(b - m2))
    lse = m2 + jnp.log(s2)
    valid = lab < _NPIDS
    terms = jnp.where(valid, iou * iou * (lse - b), 0.0)
    nvalid = jnp.sum(jnp.where(valid, 1.0, 0.0), axis=(0, 1),
                     keepdims=True)
    out_ref[...] = jnp.sum(terms, axis=(0, 1), keepdims=True) / nvalid


def kernel(inputs, label, iou, lut, cq):
    lab_safe = jnp.clip(label, 0, _NPIDS - 1)
    g = _gather_label_rows(lut, lab_safe)
    m, s, nb = pl.pallas_call(
        _stream_body,
        grid=(_K,),
        in_specs=[
            pl.BlockSpec((_B, _F), lambda k: (0, 0)),
            pl.BlockSpec((_NL_TILE, _F),
                         lambda k: (jnp.minimum(k, _KL - 1), 0)),
            pl.BlockSpec((_NC_TILE, _F),
                         lambda k: (jnp.maximum(k - _KL, 0), 0)),
        ],
        out_specs=[
            pl.BlockSpec((_B, 1), lambda k: (0, 0)),
            pl.BlockSpec((_B, 1), lambda k: (0, 0)),
            pl.BlockSpec((1, 1), lambda k: (0, 0)),
        ],
        out_shape=[
            jax.ShapeDtypeStruct((_B, 1), jnp.float32),
            jax.ShapeDtypeStruct((_B, 1), jnp.float32),
            jax.ShapeDtypeStruct((1, 1), jnp.float32),
        ],
        scratch_shapes=[
            pltpu.VMEM((_B, _F), jnp.bfloat16),
        ],
        compiler_params=pltpu.CompilerParams(
            dimension_semantics=("arbitrary",),
        ),
    )(inputs, lut, cq)
    lab2 = label.reshape(_B, 1)
    iou2 = iou.reshape(_B, 1)
    out = pl.pallas_call(
        _final_body,
        in_specs=[
            pl.BlockSpec((_B, _F), lambda: (0, 0)),
            pl.BlockSpec((_B, 1), lambda: (0, 0)),
            pl.BlockSpec((_B, 1), lambda: (0, 0)),
            pl.BlockSpec((_B, _F), lambda: (0, 0)),
            pl.BlockSpec((_B, 1), lambda: (0, 0)),
            pl.BlockSpec((_B, 1), lambda: (0, 0)),
            pl.BlockSpec((1, 1), lambda: (0, 0)),
        ],
        out_specs=pl.BlockSpec((1, 1), lambda: (0, 0)),
        out_shape=jax.ShapeDtypeStruct((1, 1), jnp.float32),
    )(inputs, lab2, iou2, g, m, s, nb)
    return out[0, 0]


# speedup vs baseline: 1.0563x; 1.0197x over previous
import jax
import jax.numpy as jnp
from jax import lax
from jax.experimental import pallas as pl
from jax.experimental.pallas import tpu as pltpu
from jax.experimental.pallas import tpu_sc as plsc

_F = 128
_NPIDS = 100000
_NCQ = 5000
_SCALE = 30.0
_B = 1024
_LOG2E = 1.4426950408889634
_LN2 = 0.6931471805599453

_NL_TILE = 10000
_KL = _NPIDS // _NL_TILE
_NC_TILE = 5000
_KC = _NCQ // _NC_TILE
_K = _KL + _KC

_NW = 32
_BPW = _B // _NW


def _sc_gather_body(table_hbm, idx_hbm, out_hbm, idx_v, rows_v, sem):
    wid = lax.axis_index("s") * 2 + lax.axis_index("c")
    base = wid * _BPW
    pltpu.sync_copy(idx_hbm.at[pl.ds(base, _BPW)], idx_v)
    pltpu.async_copy(table_hbm.at[idx_v], rows_v, sem).wait()
    pltpu.sync_copy(rows_v, out_hbm.at[pl.ds(base, _BPW)])


def _gather_label_rows(lut, idx):
    mesh = plsc.VectorSubcoreMesh(core_axis_name="c", subcore_axis_name="s")
    return pl.kernel(
        _sc_gather_body,
        mesh=mesh,
        out_type=jax.ShapeDtypeStruct((_B, _F), jnp.float32),
        scratch_types=[
            pltpu.VMEM((_BPW,), jnp.int32),
            pltpu.VMEM((_BPW, _F), jnp.float32),
            pltpu.SemaphoreType.DMA,
        ],
    )(lut, idx)


def _body(x_ref, lab_ref, iou_ref, g_ref, lut_ref, cq_ref, out_ref,
          xh_ref, xh16_ref, m_ref, s_ref, nb_ref):
    k = pl.program_id(0)

    @pl.when(k == 0)
    def _init():
        x = x_ref[...]
        ss = jnp.sum(x * x, axis=1, keepdims=True)
        nrm = jnp.maximum(jnp.sqrt(ss), 1e-12)
        xh_ref[...] = x * (_SCALE / nrm)
        xh16_ref[...] = (xh_ref[...] * _LOG2E).astype(jnp.bfloat16)
        m_ref[...] = jnp.full((_B, 1), -1e30, jnp.float32)
        s_ref[...] = jnp.zeros((_B, 1), jnp.float32)
        nb_ref[...] = jnp.zeros((1, 1), jnp.float32)

    def _process(tbl):
        tbl16 = tbl.astype(jnp.bfloat16)
        ones = jnp.ones((1, _F), jnp.bfloat16)
        q = jax.lax.dot_general(ones, tbl16 * tbl16,
                                (((1,), (1,)), ((), ())),
                                preferred_element_type=jnp.float32)
        nb_ref[...] += jnp.sum(jnp.where(q == 0.0, 1.0, 0.0),
                               axis=(0, 1), keepdims=True)
        bound = (_SCALE * _LOG2E * 1.01) * jnp.sqrt(
            jnp.max(q, axis=(0, 1), keepdims=True))
        p = jax.lax.dot_general(xh16_ref[...], tbl16,
                                (((1,), (1,)), ((), ())),
                                preferred_element_type=jnp.float32)
        m_old = m_ref[...]
        m_new = jnp.maximum(m_old, bound)
        s_ref[...] = (s_ref[...] * jnp.exp2(m_old - m_new)
                      + jnp.sum(jnp.exp2(p - m_new), axis=1, keepdims=True))
        m_ref[...] = m_new

    @pl.when(k < _KL)
    def _lut_step():
        _process(lut_ref[...])

    @pl.when(k >= _KL)
    def _cq_step():
        _process(cq_ref[...])

    @pl.when(k == _K - 1)
    def _finish():
        lab = lab_ref[...]
        iou = iou_ref[...]
        g = g_ref[...]
        xh = xh_ref[...]
        d = jnp.sum(xh * g, axis=1, keepdims=True)
        gz = jnp.sum(jnp.abs(g), axis=1, keepdims=True)
        bad_l = gz == 0.0
        a = jnp.where(bad_l, -_SCALE, d)
        b = jnp.where(bad_l, _SCALE, d)
        m_raw = m_ref[...] * _LN2
        nb = nb_ref[...]
        s_raw = ((s_ref[...] - nb * jnp.exp(jnp.minimum(-m_raw, 0.0)))
                 + nb * jnp.exp(jnp.minimum(-_SCALE - m_raw, 0.0)))
        m2 = jnp.maximum(m_raw, b)
        s2 = (s_raw * jnp.exp(m_raw - m2)
              - jnp.exp(a - m2) + jnp.exp(b - m2))
        lse = m2 + jnp.log(s2)
        valid = lab < _NPIDS
        terms = jnp.where(valid, iou * iou * (lse - b), 0.0)
        nvalid = jnp.sum(jnp.where(valid, 1.0, 0.0), axis=(0, 1),
                         keepdims=True)
        out_ref[...] = jnp.sum(terms, axis=(0, 1), keepdims=True) / nvalid


def kernel(inputs, label, iou, lut, cq):
    lab_safe = jnp.clip(label, 0, _NPIDS - 1)
    g = _gather_label_rows(lut, lab_safe)
    lab2 = label.reshape(_B, 1)
    iou2 = iou.reshape(_B, 1)
    out = pl.pallas_call(
        _body,
        grid=(_K,),
        in_specs=[
            pl.BlockSpec((_B, _F), lambda k: (0, 0)),
            pl.BlockSpec((_B, 1), lambda k: (0, 0)),
            pl.BlockSpec((_B, 1), lambda k: (0, 0)),
            pl.BlockSpec((_B, _F), lambda k: (0, 0)),
            pl.BlockSpec((_NL_TILE, _F),
                         lambda k: (jnp.minimum(k, _KL - 1), 0)),
            pl.BlockSpec((_NC_TILE, _F),
                         lambda k: (jnp.maximum(k - _KL, 0), 0)),
        ],
        out_specs=pl.BlockSpec((1, 1), lambda k: (0, 0)),
        out_shape=jax.ShapeDtypeStruct((1, 1), jnp.float32),
        scratch_shapes=[
            pltpu.VMEM((_B, _F), jnp.float32),
            pltpu.VMEM((_B, _F), jnp.bfloat16),
            pltpu.VMEM((_B, 1), jnp.float32),
            pltpu.VMEM((_B, 1), jnp.float32),
            pltpu.VMEM((1, 1), jnp.float32),
        ],
        compiler_params=pltpu.CompilerParams(
            dimension_semantics=("arbitrary",),
        ),
    )(inputs, lab2, iou2, g, lut, cq)
    return out[0, 0]
